# baseline (speedup 1.0000x reference)
import functools

import numpy as np
import jax
import jax.numpy as jnp
from jax import lax
from jax.experimental import pallas as pl
from jax.experimental.pallas import tpu as pltpu
from jax.experimental.pallas import tpu_sc as plsc

_SIGMA = 0.2
_B = 64
_ROW = 56 * 56 * 192
_NC, _NS = 2, 16
_NW = _NC * _NS
_W = _ROW // _NW
_NV = _W // 16

_PERM = (17, 27, 42, 32, 1, 3, 58, 51, 40, 28, 52, 19, 9, 33, 11, 45,
         31, 5, 15, 39, 50, 47, 20, 0, 46, 14, 49, 44, 38, 61, 2, 54,
         36, 35, 62, 63, 21, 59, 30, 43, 22, 18, 24, 26, 53, 12, 16, 6,
         7, 57, 55, 48, 13, 37, 60, 10, 29, 34, 25, 56, 4, 41, 23, 8)


def _cycles(perm):
    seen, out = [False] * len(perm), []
    for s in range(len(perm)):
        if seen[s]:
            continue
        c, j = [s], perm[s]
        seen[s] = True
        while j != s:
            c.append(j)
            seen[j] = True
            j = perm[j]
        out.append(c)
    return out


_CYCLES = _cycles(_PERM)

_cache = {}


def _coeffs():
    key = jax.random.key(42)
    _, k_r, k_theta = jax.random.split(key, 3)
    rs = (1, 56, 56, 192)
    r = jax.random.normal(k_r, rs, dtype=jnp.float16) * jnp.float16(_SIGMA)
    theta = jax.random.uniform(k_theta, rs, dtype=jnp.float16,
                               minval=-np.pi, maxval=np.pi)
    a = (jnp.float16(1.0) + r * jnp.cos(theta)).astype(jnp.float32).reshape(_ROW)
    b = (r * jnp.sin(theta)).astype(jnp.float32).reshape(_ROW)
    return a, b


def _pack_coeffs(a, b):
    a16 = lax.bitcast_convert_type(a.astype(jnp.bfloat16), jnp.uint16)
    b16 = lax.bitcast_convert_type(b.astype(jnp.bfloat16), jnp.uint16)
    packed = (b16.astype(jnp.uint32) << 16) | a16.astype(jnp.uint32)
    return lax.bitcast_convert_type(packed, jnp.int32)


def _build():
    mesh = plsc.VectorSubcoreMesh(core_axis_name="c", subcore_axis_name="s")

    ring_rows, cyc_spans = [], []
    for cyc in _CYCLES:
        cyc_spans.append((len(ring_rows), len(cyc), cyc))
        ring_rows.extend(cyc)
        ring_rows.append(cyc[0])
    n_loads = len(ring_rows)

    @functools.partial(
        pl.kernel,
        mesh=mesh,
        out_type=jax.ShapeDtypeStruct((_B * _ROW,), jnp.float32),
        scratch_types=[
            pltpu.VMEM((_W,), jnp.int32),
            pltpu.VMEM((_W,), jnp.float32),
            pltpu.VMEM((_W,), jnp.float32),
            pltpu.VMEM((_W,), jnp.float32),
            pltpu.VMEM((_W,), jnp.float32),
            pltpu.SemaphoreType.DMA,
            pltpu.SemaphoreType.DMA,
            pltpu.SemaphoreType.DMA,
            pltpu.SemaphoreType.DMA,
        ],
    )
    def mixfeat(x_hbm, c_hbm, y_hbm, c_v, xr0, xr1, or0, or1, xs0, xs1, os0, os1):
        wid = lax.axis_index("s") * _NC + lax.axis_index("c")
        base = wid * _W
        xr, orr, xsem, osem = (xr0, xr1), (or0, or1), (xs0, xs1), (os0, os1)

        pltpu.sync_copy(c_hbm.at[pl.ds(base, _W)], c_v)

        def compute(xa_ref, xb_ref, o_ref):
            @plsc.parallel_loop(0, _W, 16, unroll=8)
            def body(v):
                s = pl.ds(v, 16)
                cc = c_v[s]
                av = lax.bitcast_convert_type(lax.shift_left(cc, 16),
                                              jnp.float32)
                bv = lax.bitcast_convert_type(
                    lax.bitwise_and(cc, jnp.int32(-65536)), jnp.float32)
                o_ref[s] = xa_ref[s] * av + xb_ref[s] * bv

        load_h = [None] * n_loads
        load_waited = [False] * n_loads

        def issue_load(li):
            if li < n_loads:
                load_h[li] = pltpu.async_copy(
                    x_hbm.at[pl.ds(ring_rows[li] * _ROW + base, _W)],
                    xr[li % 2], xsem[li % 2])

        def wait_load(li):
            if not load_waited[li]:
                load_h[li].wait()
                load_waited[li] = True

        issue_load(0)
        issue_load(1)
        store_h = [None, None]
        q = 0
        for start, m, cyc in cyc_spans:
            for k in range(m):
                a_li, b_li = start + k, start + k + 1
                wait_load(a_li)
                wait_load(b_li)
                if store_h[q % 2] is not None:
                    store_h[q % 2].wait()
                compute(xr[a_li % 2], xr[b_li % 2], orr[q % 2])
                store_h[q % 2] = pltpu.async_copy(
                    orr[q % 2],
                    y_hbm.at[pl.ds(cyc[k] * _ROW + base, _W)],
                    osem[q % 2])
                if k < m - 1:
                    issue_load(a_li + 2)
                else:
                    issue_load(start + m + 1)
                    issue_load(start + m + 2)
                q += 1
        store_h[0].wait()
        store_h[1].wait()

    return mixfeat


def kernel(inputs):
    return kernel_tc(inputs)


def kernel_sc_unused(inputs):
    if "f" not in _cache:
        _cache["f"] = _build()
    a, b = _coeffs()
    c = _pack_coeffs(a, b)
    x = inputs.reshape(_B * _ROW)
    y = _cache["f"](x, c)
    return y.reshape(inputs.shape)


_TCC = 4704
_TCB = 784


def _build_tc():
    grid = (_TCC // _TCB, _B)

    def body(perm_ref, xs_ref, xp_ref, a_ref, b_ref, o_ref):
        o_ref[...] = xs_ref[...] * a_ref[...] + xp_ref[...] * b_ref[...]

    return pl.pallas_call(
        body,
        grid_spec=pltpu.PrefetchScalarGridSpec(
            num_scalar_prefetch=1,
            grid=grid,
            in_specs=[
                pl.BlockSpec((1, _TCB, 128), lambda j, i, pref: (i, j, 0)),
                pl.BlockSpec((1, _TCB, 128), lambda j, i, pref: (pref[i], j, 0)),
                pl.BlockSpec((_TCB, 128), lambda j, i, pref: (j, 0)),
                pl.BlockSpec((_TCB, 128), lambda j, i, pref: (j, 0)),
            ],
            out_specs=pl.BlockSpec((1, _TCB, 128), lambda j, i, pref: (i, j, 0)),
        ),
        out_shape=jax.ShapeDtypeStruct((_B, _TCC, 128), jnp.float32),
        compiler_params=pltpu.CompilerParams(
            dimension_semantics=("arbitrary", "arbitrary")),
    )


def kernel_tc(inputs):
    if "tc" not in _cache:
        _cache["tc"] = _build_tc()
    a, b = _coeffs()
    x3 = inputs.reshape(_B, _TCC, 128)
    perm_arr = jnp.asarray(_PERM, dtype=jnp.int32)
    y = _cache["tc"](perm_arr, x3, x3,
                     a.reshape(_TCC, 128), b.reshape(_TCC, 128))
    return y.reshape(inputs.shape)


# speedup vs baseline: 1.7523x; 1.7523x over previous
import functools

import numpy as np
import jax
import jax.numpy as jnp
from jax import lax
from jax.experimental import pallas as pl
from jax.experimental.pallas import tpu as pltpu
from jax.experimental.pallas import tpu_sc as plsc

_SIGMA = 0.2
_B = 64
_ROW = 56 * 56 * 192
_NC, _NS = 2, 16
_NW = _NC * _NS
_W = _ROW // _NW
_NV = _W // 16

_PERM = (17, 27, 42, 32, 1, 3, 58, 51, 40, 28, 52, 19, 9, 33, 11, 45,
         31, 5, 15, 39, 50, 47, 20, 0, 46, 14, 49, 44, 38, 61, 2, 54,
         36, 35, 62, 63, 21, 59, 30, 43, 22, 18, 24, 26, 53, 12, 16, 6,
         7, 57, 55, 48, 13, 37, 60, 10, 29, 34, 25, 56, 4, 41, 23, 8)


def _cycles(perm):
    seen, out = [False] * len(perm), []
    for s in range(len(perm)):
        if seen[s]:
            continue
        c, j = [s], perm[s]
        seen[s] = True
        while j != s:
            c.append(j)
            seen[j] = True
            j = perm[j]
        out.append(c)
    return out


_CYCLES = _cycles(_PERM)

_cache = {}


def _coeffs():
    key = jax.random.key(42)
    _, k_r, k_theta = jax.random.split(key, 3)
    rs = (1, 56, 56, 192)
    r = jax.random.normal(k_r, rs, dtype=jnp.float16) * jnp.float16(_SIGMA)
    theta = jax.random.uniform(k_theta, rs, dtype=jnp.float16,
                               minval=-np.pi, maxval=np.pi)
    a = (jnp.float16(1.0) + r * jnp.cos(theta)).astype(jnp.float32).reshape(_ROW)
    b = (r * jnp.sin(theta)).astype(jnp.float32).reshape(_ROW)
    return a, b


def _pack_coeffs(a, b):
    a16 = lax.bitcast_convert_type(a.astype(jnp.bfloat16), jnp.uint16)
    b16 = lax.bitcast_convert_type(b.astype(jnp.bfloat16), jnp.uint16)
    packed = (b16.astype(jnp.uint32) << 16) | a16.astype(jnp.uint32)
    return lax.bitcast_convert_type(packed, jnp.int32)


def _build():
    mesh = plsc.VectorSubcoreMesh(core_axis_name="c", subcore_axis_name="s")

    ring_rows, cyc_spans = [], []
    for cyc in _CYCLES:
        cyc_spans.append((len(ring_rows), len(cyc), cyc))
        ring_rows.extend(cyc)
        ring_rows.append(cyc[0])
    n_loads = len(ring_rows)

    @functools.partial(
        pl.kernel,
        mesh=mesh,
        out_type=jax.ShapeDtypeStruct((_B * _ROW,), jnp.float32),
        scratch_types=[
            pltpu.VMEM((_W,), jnp.int32),
            pltpu.VMEM((_W,), jnp.float32),
            pltpu.VMEM((_W,), jnp.float32),
            pltpu.VMEM((_W,), jnp.float32),
            pltpu.VMEM((_W,), jnp.float32),
            pltpu.SemaphoreType.DMA,
            pltpu.SemaphoreType.DMA,
            pltpu.SemaphoreType.DMA,
            pltpu.SemaphoreType.DMA,
        ],
    )
    def mixfeat(x_hbm, c_hbm, y_hbm, c_v, xr0, xr1, or0, or1, xs0, xs1, os0, os1):
        wid = lax.axis_index("s") * _NC + lax.axis_index("c")
        base = wid * _W
        xr, orr, xsem, osem = (xr0, xr1), (or0, or1), (xs0, xs1), (os0, os1)

        pltpu.sync_copy(c_hbm.at[pl.ds(base, _W)], c_v)

        def compute(xa_ref, xb_ref, o_ref):
            @plsc.parallel_loop(0, _W, 16, unroll=8)
            def body(v):
                s = pl.ds(v, 16)
                cc = c_v[s]
                av = lax.bitcast_convert_type(lax.shift_left(cc, 16),
                                              jnp.float32)
                bv = lax.bitcast_convert_type(
                    lax.bitwise_and(cc, jnp.int32(-65536)), jnp.float32)
                o_ref[s] = xa_ref[s] * av + xb_ref[s] * bv

        load_h = [None] * n_loads
        load_waited = [False] * n_loads

        def issue_load(li):
            if li < n_loads:
                load_h[li] = pltpu.async_copy(
                    x_hbm.at[pl.ds(ring_rows[li] * _ROW + base, _W)],
                    xr[li % 2], xsem[li % 2])

        def wait_load(li):
            if not load_waited[li]:
                load_h[li].wait()
                load_waited[li] = True

        issue_load(0)
        issue_load(1)
        store_h = [None, None]
        q = 0
        for start, m, cyc in cyc_spans:
            for k in range(m):
                a_li, b_li = start + k, start + k + 1
                wait_load(a_li)
                wait_load(b_li)
                if store_h[q % 2] is not None:
                    store_h[q % 2].wait()
                compute(xr[a_li % 2], xr[b_li % 2], orr[q % 2])
                store_h[q % 2] = pltpu.async_copy(
                    orr[q % 2],
                    y_hbm.at[pl.ds(cyc[k] * _ROW + base, _W)],
                    osem[q % 2])
                if k < m - 1:
                    issue_load(a_li + 2)
                else:
                    issue_load(start + m + 1)
                    issue_load(start + m + 2)
                q += 1
        store_h[0].wait()
        store_h[1].wait()

    return mixfeat


def kernel(inputs):
    return kernel_tc(inputs)


def kernel_sc_unused(inputs):
    if "f" not in _cache:
        _cache["f"] = _build()
    a, b = _coeffs()
    c = _pack_coeffs(a, b)
    x = inputs.reshape(_B * _ROW)
    y = _cache["f"](x, c)
    return y.reshape(inputs.shape)


def _build_tc():
    grid = (7, _B)

    def body(perm_ref, xs_ref, xp_ref, a_ref, b_ref, o_ref):
        o_ref[...] = xs_ref[...] * a_ref[...] + xp_ref[...] * b_ref[...]

    blk = (1, 8, 56, 192)
    return pl.pallas_call(
        body,
        grid_spec=pltpu.PrefetchScalarGridSpec(
            num_scalar_prefetch=1,
            grid=grid,
            in_specs=[
                pl.BlockSpec(blk, lambda j, i, pref: (i, j, 0, 0)),
                pl.BlockSpec(blk, lambda j, i, pref: (pref[i], j, 0, 0)),
                pl.BlockSpec((8, 56, 192), lambda j, i, pref: (j, 0, 0)),
                pl.BlockSpec((8, 56, 192), lambda j, i, pref: (j, 0, 0)),
            ],
            out_specs=pl.BlockSpec(blk, lambda j, i, pref: (i, j, 0, 0)),
        ),
        out_shape=jax.ShapeDtypeStruct((_B, 56, 56, 192), jnp.float32),
        compiler_params=pltpu.CompilerParams(
            dimension_semantics=("arbitrary", "arbitrary")),
    )


def kernel_tc(inputs):
    if "tc" not in _cache:
        _cache["tc"] = _build_tc()
    a, b = _coeffs()
    perm_arr = jnp.asarray(_PERM, dtype=jnp.int32)
    y = _cache["tc"](perm_arr, inputs, inputs,
                     a.reshape(56, 56, 192), b.reshape(56, 56, 192))
    return y


# speedup vs baseline: 1.8147x; 1.0356x over previous
import functools

import numpy as np
import jax
import jax.numpy as jnp
from jax import lax
from jax.experimental import pallas as pl
from jax.experimental.pallas import tpu as pltpu
from jax.experimental.pallas import tpu_sc as plsc

_SIGMA = 0.2
_B = 64
_ROW = 56 * 56 * 192
_NC, _NS = 2, 16
_NW = _NC * _NS
_W = _ROW // _NW
_NV = _W // 16

_PERM = (17, 27, 42, 32, 1, 3, 58, 51, 40, 28, 52, 19, 9, 33, 11, 45,
         31, 5, 15, 39, 50, 47, 20, 0, 46, 14, 49, 44, 38, 61, 2, 54,
         36, 35, 62, 63, 21, 59, 30, 43, 22, 18, 24, 26, 53, 12, 16, 6,
         7, 57, 55, 48, 13, 37, 60, 10, 29, 34, 25, 56, 4, 41, 23, 8)


def _cycles(perm):
    seen, out = [False] * len(perm), []
    for s in range(len(perm)):
        if seen[s]:
            continue
        c, j = [s], perm[s]
        seen[s] = True
        while j != s:
            c.append(j)
            seen[j] = True
            j = perm[j]
        out.append(c)
    return out


_CYCLES = _cycles(_PERM)

_cache = {}


def _coeffs():
    key = jax.random.key(42)
    _, k_r, k_theta = jax.random.split(key, 3)
    rs = (1, 56, 56, 192)
    r = jax.random.normal(k_r, rs, dtype=jnp.float16) * jnp.float16(_SIGMA)
    theta = jax.random.uniform(k_theta, rs, dtype=jnp.float16,
                               minval=-np.pi, maxval=np.pi)
    a = (jnp.float16(1.0) + r * jnp.cos(theta)).astype(jnp.float32).reshape(_ROW)
    b = (r * jnp.sin(theta)).astype(jnp.float32).reshape(_ROW)
    return a, b


def _pack_coeffs(a, b):
    a16 = lax.bitcast_convert_type(a.astype(jnp.bfloat16), jnp.uint16)
    b16 = lax.bitcast_convert_type(b.astype(jnp.bfloat16), jnp.uint16)
    packed = (b16.astype(jnp.uint32) << 16) | a16.astype(jnp.uint32)
    return lax.bitcast_convert_type(packed, jnp.int32)


def _build():
    mesh = plsc.VectorSubcoreMesh(core_axis_name="c", subcore_axis_name="s")

    ring_rows, cyc_spans = [], []
    for cyc in _CYCLES:
        cyc_spans.append((len(ring_rows), len(cyc), cyc))
        ring_rows.extend(cyc)
        ring_rows.append(cyc[0])
    n_loads = len(ring_rows)

    @functools.partial(
        pl.kernel,
        mesh=mesh,
        out_type=jax.ShapeDtypeStruct((_B * _ROW,), jnp.float32),
        scratch_types=[
            pltpu.VMEM((_W,), jnp.int32),
            pltpu.VMEM((_W,), jnp.float32),
            pltpu.VMEM((_W,), jnp.float32),
            pltpu.VMEM((_W,), jnp.float32),
            pltpu.VMEM((_W,), jnp.float32),
            pltpu.SemaphoreType.DMA,
            pltpu.SemaphoreType.DMA,
            pltpu.SemaphoreType.DMA,
            pltpu.SemaphoreType.DMA,
        ],
    )
    def mixfeat(x_hbm, c_hbm, y_hbm, c_v, xr0, xr1, or0, or1, xs0, xs1, os0, os1):
        wid = lax.axis_index("s") * _NC + lax.axis_index("c")
        base = wid * _W
        xr, orr, xsem, osem = (xr0, xr1), (or0, or1), (xs0, xs1), (os0, os1)

        pltpu.sync_copy(c_hbm.at[pl.ds(base, _W)], c_v)

        def compute(xa_ref, xb_ref, o_ref):
            @plsc.parallel_loop(0, _W, 16, unroll=8)
            def body(v):
                s = pl.ds(v, 16)
                cc = c_v[s]
                av = lax.bitcast_convert_type(lax.shift_left(cc, 16),
                                              jnp.float32)
                bv = lax.bitcast_convert_type(
                    lax.bitwise_and(cc, jnp.int32(-65536)), jnp.float32)
                o_ref[s] = xa_ref[s] * av + xb_ref[s] * bv

        load_h = [None] * n_loads
        load_waited = [False] * n_loads

        def issue_load(li):
            if li < n_loads:
                load_h[li] = pltpu.async_copy(
                    x_hbm.at[pl.ds(ring_rows[li] * _ROW + base, _W)],
                    xr[li % 2], xsem[li % 2])

        def wait_load(li):
            if not load_waited[li]:
                load_h[li].wait()
                load_waited[li] = True

        issue_load(0)
        issue_load(1)
        store_h = [None, None]
        q = 0
        for start, m, cyc in cyc_spans:
            for k in range(m):
                a_li, b_li = start + k, start + k + 1
                wait_load(a_li)
                wait_load(b_li)
                if store_h[q % 2] is not None:
                    store_h[q % 2].wait()
                compute(xr[a_li % 2], xr[b_li % 2], orr[q % 2])
                store_h[q % 2] = pltpu.async_copy(
                    orr[q % 2],
                    y_hbm.at[pl.ds(cyc[k] * _ROW + base, _W)],
                    osem[q % 2])
                if k < m - 1:
                    issue_load(a_li + 2)
                else:
                    issue_load(start + m + 1)
                    issue_load(start + m + 2)
                q += 1
        store_h[0].wait()
        store_h[1].wait()

    return mixfeat


def kernel(inputs):
    return kernel_tc(inputs)


def kernel_sc_unused(inputs):
    if "f" not in _cache:
        _cache["f"] = _build()
    a, b = _coeffs()
    c = _pack_coeffs(a, b)
    x = inputs.reshape(_B * _ROW)
    y = _cache["f"](x, c)
    return y.reshape(inputs.shape)


def _schedule():
    srow, orow, flag = [], [], []
    for cyc in _CYCLES:
        for t, r in enumerate([*cyc, cyc[0]]):
            srow.append(r)
            if t == 0:
                orow.append(cyc[0])
                flag.append(0)
            else:
                orow.append(cyc[t - 1])
                flag.append(1)
    return (np.asarray(srow, np.int32), np.asarray(orow, np.int32),
            np.asarray(flag, np.int32))


_SROW, _OROW, _FLAG = _schedule()
_NSTEPS = len(_SROW)


def _build_tc():
    grid = (7, _NSTEPS)
    blk = (1, 8, 56, 192)

    def body(srow_ref, orow_ref, flag_ref, xs_ref, a_ref, b_ref, o_ref,
             prev_ref):
        k = pl.program_id(1)

        @pl.when(flag_ref[k] != 0)
        def _():
            o_ref[0] = (prev_ref[...] * a_ref[...]
                        + xs_ref[0] * b_ref[...])

        prev_ref[...] = xs_ref[0]

    return pl.pallas_call(
        body,
        grid_spec=pltpu.PrefetchScalarGridSpec(
            num_scalar_prefetch=3,
            grid=grid,
            in_specs=[
                pl.BlockSpec(blk, lambda j, k, sr, orr, fl: (sr[k], j, 0, 0)),
                pl.BlockSpec((8, 56, 192), lambda j, k, sr, orr, fl: (j, 0, 0)),
                pl.BlockSpec((8, 56, 192), lambda j, k, sr, orr, fl: (j, 0, 0)),
            ],
            out_specs=pl.BlockSpec(blk, lambda j, k, sr, orr, fl: (orr[k], j, 0, 0)),
            scratch_shapes=[pltpu.VMEM((8, 56, 192), jnp.float32)],
        ),
        out_shape=jax.ShapeDtypeStruct((_B, 56, 56, 192), jnp.float32),
        compiler_params=pltpu.CompilerParams(
            dimension_semantics=("arbitrary", "arbitrary")),
    )


def kernel_tc(inputs):
    if "tc" not in _cache:
        _cache["tc"] = _build_tc()
    a, b = _coeffs()
    y = _cache["tc"](jnp.asarray(_SROW), jnp.asarray(_OROW), jnp.asarray(_FLAG),
                     inputs, a.reshape(56, 56, 192), b.reshape(56, 56, 192))
    return y


# speedup vs baseline: 3.4090x; 1.8786x over previous
import functools

import numpy as np
import jax
import jax.numpy as jnp
from jax import lax
from jax.experimental import pallas as pl
from jax.experimental.pallas import tpu as pltpu
from jax.experimental.pallas import tpu_sc as plsc

_SIGMA = 0.2
_B = 64
_ROW = 56 * 56 * 192
_NC, _NS = 2, 16
_NW = _NC * _NS
_W = _ROW // _NW
_NV = _W // 16

_PERM = (17, 27, 42, 32, 1, 3, 58, 51, 40, 28, 52, 19, 9, 33, 11, 45,
         31, 5, 15, 39, 50, 47, 20, 0, 46, 14, 49, 44, 38, 61, 2, 54,
         36, 35, 62, 63, 21, 59, 30, 43, 22, 18, 24, 26, 53, 12, 16, 6,
         7, 57, 55, 48, 13, 37, 60, 10, 29, 34, 25, 56, 4, 41, 23, 8)


def _cycles(perm):
    seen, out = [False] * len(perm), []
    for s in range(len(perm)):
        if seen[s]:
            continue
        c, j = [s], perm[s]
        seen[s] = True
        while j != s:
            c.append(j)
            seen[j] = True
            j = perm[j]
        out.append(c)
    return out


_CYCLES = _cycles(_PERM)

_cache = {}


def _coeffs():
    key = jax.random.key(42)
    _, k_r, k_theta = jax.random.split(key, 3)
    rs = (1, 56, 56, 192)
    r = jax.random.normal(k_r, rs, dtype=jnp.float16) * jnp.float16(_SIGMA)
    theta = jax.random.uniform(k_theta, rs, dtype=jnp.float16,
                               minval=-np.pi, maxval=np.pi)
    a = (jnp.float16(1.0) + r * jnp.cos(theta)).astype(jnp.float32).reshape(_ROW)
    b = (r * jnp.sin(theta)).astype(jnp.float32).reshape(_ROW)
    return a, b


def _pack_coeffs(a, b):
    a16 = lax.bitcast_convert_type(a.astype(jnp.bfloat16), jnp.uint16)
    b16 = lax.bitcast_convert_type(b.astype(jnp.bfloat16), jnp.uint16)
    packed = (b16.astype(jnp.uint32) << 16) | a16.astype(jnp.uint32)
    return lax.bitcast_convert_type(packed, jnp.int32)


def _build():
    mesh = plsc.VectorSubcoreMesh(core_axis_name="c", subcore_axis_name="s")

    ring_rows, cyc_spans = [], []
    for cyc in _CYCLES:
        cyc_spans.append((len(ring_rows), len(cyc), cyc))
        ring_rows.extend(cyc)
        ring_rows.append(cyc[0])
    n_loads = len(ring_rows)

    @functools.partial(
        pl.kernel,
        mesh=mesh,
        out_type=jax.ShapeDtypeStruct((_B * _ROW,), jnp.float32),
        scratch_types=[
            pltpu.VMEM((_W,), jnp.int32),
            pltpu.VMEM((_W,), jnp.float32),
            pltpu.VMEM((_W,), jnp.float32),
            pltpu.VMEM((_W,), jnp.float32),
            pltpu.VMEM((_W,), jnp.float32),
            pltpu.SemaphoreType.DMA,
            pltpu.SemaphoreType.DMA,
            pltpu.SemaphoreType.DMA,
            pltpu.SemaphoreType.DMA,
        ],
    )
    def mixfeat(x_hbm, c_hbm, y_hbm, c_v, xr0, xr1, or0, or1, xs0, xs1, os0, os1):
        wid = lax.axis_index("s") * _NC + lax.axis_index("c")
        base = wid * _W
        xr, orr, xsem, osem = (xr0, xr1), (or0, or1), (xs0, xs1), (os0, os1)

        pltpu.sync_copy(c_hbm.at[pl.ds(base, _W)], c_v)

        def compute(xa_ref, xb_ref, o_ref):
            @plsc.parallel_loop(0, _W, 16, unroll=8)
            def body(v):
                s = pl.ds(v, 16)
                cc = c_v[s]
                av = lax.bitcast_convert_type(lax.shift_left(cc, 16),
                                              jnp.float32)
                bv = lax.bitcast_convert_type(
                    lax.bitwise_and(cc, jnp.int32(-65536)), jnp.float32)
                o_ref[s] = xa_ref[s] * av + xb_ref[s] * bv

        load_h = [None] * n_loads
        load_waited = [False] * n_loads

        def issue_load(li):
            if li < n_loads:
                load_h[li] = pltpu.async_copy(
                    x_hbm.at[pl.ds(ring_rows[li] * _ROW + base, _W)],
                    xr[li % 2], xsem[li % 2])

        def wait_load(li):
            if not load_waited[li]:
                load_h[li].wait()
                load_waited[li] = True

        issue_load(0)
        issue_load(1)
        store_h = [None, None]
        q = 0
        for start, m, cyc in cyc_spans:
            for k in range(m):
                a_li, b_li = start + k, start + k + 1
                wait_load(a_li)
                wait_load(b_li)
                if store_h[q % 2] is not None:
                    store_h[q % 2].wait()
                compute(xr[a_li % 2], xr[b_li % 2], orr[q % 2])
                store_h[q % 2] = pltpu.async_copy(
                    orr[q % 2],
                    y_hbm.at[pl.ds(cyc[k] * _ROW + base, _W)],
                    osem[q % 2])
                if k < m - 1:
                    issue_load(a_li + 2)
                else:
                    issue_load(start + m + 1)
                    issue_load(start + m + 2)
                q += 1
        store_h[0].wait()
        store_h[1].wait()

    return mixfeat


def kernel(inputs):
    return kernel_tc(inputs)


def kernel_sc_unused(inputs):
    if "f" not in _cache:
        _cache["f"] = _build()
    a, b = _coeffs()
    c = _pack_coeffs(a, b)
    x = inputs.reshape(_B * _ROW)
    y = _cache["f"](x, c)
    return y.reshape(inputs.shape)


def _schedule():
    srow, orow, flag = [], [], []
    for cyc in _CYCLES:
        for t, r in enumerate([*cyc, cyc[0]]):
            srow.append(r)
            if t == 0:
                orow.append(cyc[0])
                flag.append(0)
            else:
                orow.append(cyc[t - 1])
                flag.append(1)
    return (np.asarray(srow, np.int32), np.asarray(orow, np.int32),
            np.asarray(flag, np.int32))


_SROW, _OROW, _FLAG = _schedule()
_NSTEPS = len(_SROW)


def _build_tc():
    grid = (_NSTEPS,)
    blk = (1, 56, 56, 192)

    def body(srow_ref, orow_ref, flag_ref, xs_ref, a_ref, b_ref, o_ref,
             prev_ref):
        k = pl.program_id(0)

        @pl.when(flag_ref[k] != 0)
        def _():
            o_ref[0] = prev_ref[...] * a_ref[...] + xs_ref[0] * b_ref[...]

        prev_ref[...] = xs_ref[0]

    return pl.pallas_call(
        body,
        grid_spec=pltpu.PrefetchScalarGridSpec(
            num_scalar_prefetch=3,
            grid=grid,
            in_specs=[
                pl.BlockSpec(blk, lambda k, sr, orr, fl: (sr[k], 0, 0, 0)),
                pl.BlockSpec((56, 56, 192), lambda k, sr, orr, fl: (0, 0, 0)),
                pl.BlockSpec((56, 56, 192), lambda k, sr, orr, fl: (0, 0, 0)),
            ],
            out_specs=pl.BlockSpec(blk, lambda k, sr, orr, fl: (orr[k], 0, 0, 0)),
            scratch_shapes=[pltpu.VMEM((56, 56, 192), jnp.float32)],
        ),
        out_shape=jax.ShapeDtypeStruct((_B, 56, 56, 192), jnp.float32),
        compiler_params=pltpu.CompilerParams(
            dimension_semantics=("arbitrary",)),
    )


def kernel_tc(inputs):
    if "tc" not in _cache:
        _cache["tc"] = _build_tc()
    a, b = _coeffs()
    y = _cache["tc"](jnp.asarray(_SROW), jnp.asarray(_OROW), jnp.asarray(_FLAG),
                     inputs, a.reshape(56, 56, 192), b.reshape(56, 56, 192))
    return y


# speedup vs baseline: 3.4147x; 1.0017x over previous
import numpy as np
import jax
import jax.numpy as jnp
from jax.experimental import pallas as pl
from jax.experimental.pallas import tpu as pltpu

_SIGMA = 0.2
_B = 64

_PERM = (17, 27, 42, 32, 1, 3, 58, 51, 40, 28, 52, 19, 9, 33, 11, 45,
         31, 5, 15, 39, 50, 47, 20, 0, 46, 14, 49, 44, 38, 61, 2, 54,
         36, 35, 62, 63, 21, 59, 30, 43, 22, 18, 24, 26, 53, 12, 16, 6,
         7, 57, 55, 48, 13, 37, 60, 10, 29, 34, 25, 56, 4, 41, 23, 8)


def _cycles(perm):
    seen, out = [False] * len(perm), []
    for s in range(len(perm)):
        if seen[s]:
            continue
        c, j = [s], perm[s]
        seen[s] = True
        while j != s:
            c.append(j)
            seen[j] = True
            j = perm[j]
        out.append(c)
    return out


def _schedule():
    srow, orow, flag = [], [], []
    for cyc in _cycles(_PERM):
        for t, r in enumerate([*cyc, cyc[0]]):
            srow.append(r)
            if t == 0:
                orow.append(cyc[0])
                flag.append(0)
            else:
                orow.append(cyc[t - 1])
                flag.append(1)
    return (np.asarray(srow, np.int32), np.asarray(orow, np.int32),
            np.asarray(flag, np.int32))


_SROW, _OROW, _FLAG = _schedule()
_NSTEPS = len(_SROW)

_cache = {}


def _coeffs():
    key = jax.random.key(42)
    _, k_r, k_theta = jax.random.split(key, 3)
    rs = (1, 56, 56, 192)
    r = jax.random.normal(k_r, rs, dtype=jnp.float16) * jnp.float16(_SIGMA)
    theta = jax.random.uniform(k_theta, rs, dtype=jnp.float16,
                               minval=-np.pi, maxval=np.pi)
    a = (jnp.float16(1.0) + r * jnp.cos(theta)).astype(jnp.float32)
    b = (r * jnp.sin(theta)).astype(jnp.float32)
    return a.reshape(56, 56, 192), b.reshape(56, 56, 192)


def _build():
    grid = (_NSTEPS,)
    blk = (1, 56, 56, 192)

    def body(srow_ref, orow_ref, flag_ref, xs_ref, a_ref, b_ref, o_ref,
             prev_ref):
        k = pl.program_id(0)

        @pl.when(flag_ref[k] != 0)
        def _():
            o_ref[0] = prev_ref[...] * a_ref[...] + xs_ref[0] * b_ref[...]

        prev_ref[...] = xs_ref[0]

    return pl.pallas_call(
        body,
        grid_spec=pltpu.PrefetchScalarGridSpec(
            num_scalar_prefetch=3,
            grid=grid,
            in_specs=[
                pl.BlockSpec(blk, lambda k, sr, orr, fl: (sr[k], 0, 0, 0)),
                pl.BlockSpec((56, 56, 192), lambda k, sr, orr, fl: (0, 0, 0)),
                pl.BlockSpec((56, 56, 192), lambda k, sr, orr, fl: (0, 0, 0)),
            ],
            out_specs=pl.BlockSpec(blk,
                                   lambda k, sr, orr, fl: (orr[k], 0, 0, 0)),
            scratch_shapes=[pltpu.VMEM((56, 56, 192), jnp.float32)],
        ),
        out_shape=jax.ShapeDtypeStruct((_B, 56, 56, 192), jnp.float32),
        compiler_params=pltpu.CompilerParams(
            dimension_semantics=("arbitrary",)),
    )


def kernel(inputs):
    if "f" not in _cache:
        _cache["f"] = _build()
    a, b = _coeffs()
    return _cache["f"](jnp.asarray(_SROW), jnp.asarray(_OROW),
                       jnp.asarray(_FLAG), inputs, a, b)
